# baseline (speedup 1.0000x reference)
import functools
import math

import jax
import jax.numpy as jnp
from jax import lax
from jax.experimental import pallas as pl
from jax.experimental.pallas import tpu as pltpu
from jax.experimental.pallas import tpu_sc as plsc

D_MODEL = 64
SCALE = math.sqrt(D_MODEL)

NC = 2
NS = 16
NW = NC * NS

B_TOTAL = 4096 * 200
ROWS_PER_W = B_TOTAL // NW

K = 8
C = K * 128
CHUNKS_PER_W = ROWS_PER_W // C
IDXROWS_PER_W = ROWS_PER_W // 128

_mesh = plsc.VectorSubcoreMesh(core_axis_name="c", subcore_axis_name="s")


@functools.partial(
    pl.kernel,
    out_type=jax.ShapeDtypeStruct((B_TOTAL, D_MODEL), jnp.float32),
    mesh=_mesh,
    scratch_types=[
        pltpu.VMEM((K, 128), jnp.int32),
        pltpu.VMEM((C, D_MODEL), jnp.float32),
        pltpu.SemaphoreType.DMA,
    ],
    compiler_params=pltpu.CompilerParams(use_tc_tiling_on_sc=False),
)
def _emb_lookup(x_hbm, table_hbm, out_hbm, idx_v, buf_v, gsem):
    wid = lax.axis_index("s") * NC + lax.axis_index("c")

    def chunk(g, carry):
        ibase = wid * IDXROWS_PER_W + g * K
        pltpu.sync_copy(x_hbm.at[pl.ds(ibase, K)], idx_v)
        copies = [
            pltpu.async_copy(
                table_hbm.at[idx_v.at[j]],
                buf_v.at[pl.ds(j * 128, 128)],
                gsem,
            )
            for j in range(K)
        ]
        for cp in copies:
            cp.wait()

        def rowfn(i, c2):
            for d in range(D_MODEL // 16):
                buf_v[i, pl.ds(d * 16, 16)] = buf_v[i, pl.ds(d * 16, 16)] * SCALE
            return c2

        lax.fori_loop(0, C, rowfn, 0)

        rbase = wid * ROWS_PER_W + g * C
        pltpu.sync_copy(buf_v, out_hbm.at[pl.ds(rbase, C)])
        return carry

    lax.fori_loop(0, CHUNKS_PER_W, chunk, 0)


def kernel(x, table):
    xf = x.astype(jnp.int32).reshape(B_TOTAL // 128, 128)
    out = _emb_lookup(xf, table)
    return out.reshape(x.shape[0], x.shape[1], D_MODEL)


# speedup vs baseline: 1.1060x; 1.1060x over previous
import functools
import math

import jax
import jax.numpy as jnp
from jax import lax
from jax.experimental import pallas as pl
from jax.experimental.pallas import tpu as pltpu
from jax.experimental.pallas import tpu_sc as plsc

D_MODEL = 64
SCALE = math.sqrt(D_MODEL)

NC = 2
NS = 16
NW = NC * NS

B_TOTAL = 4096 * 200
ROWS_PER_W = B_TOTAL // NW
IDXW = 128
IDXR_PER_W = ROWS_PER_W // IDXW

RPC = 5
C = RPC * IDXW
NCH = ROWS_PER_W // C

_mesh = plsc.VectorSubcoreMesh(core_axis_name="c", subcore_axis_name="s")


@functools.partial(
    pl.kernel,
    out_type=jax.ShapeDtypeStruct((B_TOTAL, D_MODEL), jnp.float32),
    mesh=_mesh,
    scratch_types=[
        pltpu.VMEM((IDXR_PER_W, IDXW), jnp.int32),
        pltpu.VMEM((C, D_MODEL), jnp.float32),
        pltpu.VMEM((C, D_MODEL), jnp.float32),
        pltpu.SemaphoreType.DMA,
        pltpu.SemaphoreType.DMA,
        pltpu.SemaphoreType.DMA,
        pltpu.SemaphoreType.DMA,
    ],
    compiler_params=pltpu.CompilerParams(use_tc_tiling_on_sc=False),
)
def _emb_lookup(x_hbm, table_hbm, out_hbm, idx_all, buf0, buf1,
                gsem0, gsem1, ssem0, ssem1):
    wid = lax.axis_index("s") * NC + lax.axis_index("c")
    obase = wid * ROWS_PER_W

    bufs = (buf0, buf1)
    gsems = (gsem0, gsem1)
    ssems = (ssem0, ssem1)

    pltpu.sync_copy(x_hbm.at[pl.ds(wid * IDXR_PER_W, IDXR_PER_W)], idx_all)

    def fire_gather(h, b):
        for j in range(RPC):
            pltpu.async_copy(
                table_hbm.at[idx_all.at[h * RPC + j]],
                bufs[b].at[pl.ds(j * IDXW, IDXW)],
                gsems[b],
            )

    def wait_gather(h, b):
        for j in range(RPC):
            pltpu.make_async_copy(
                table_hbm.at[idx_all.at[h * RPC + j]],
                bufs[b].at[pl.ds(j * IDXW, IDXW)],
                gsems[b],
            ).wait()

    def scale(b):
        buf = bufs[b]

        @plsc.parallel_loop(0, C, step=1, unroll=8)
        def _(i):
            for d in range(D_MODEL // 16):
                buf[i, pl.ds(d * 16, 16)] = buf[i, pl.ds(d * 16, 16)] * SCALE

    def fire_store(h, b):
        pltpu.async_copy(bufs[b], out_hbm.at[pl.ds(obase + h * C, C)], ssems[b])

    def wait_store(h, b):
        pltpu.make_async_copy(
            bufs[b], out_hbm.at[pl.ds(obase + h * C, C)], ssems[b]
        ).wait()

    fire_gather(0, 0)
    fire_gather(1, 1)

    @pl.loop(0, NCH, step=2)
    def _(g):
        for b in (0, 1):
            h = g + b
            wait_gather(h, b)
            scale(b)
            fire_store(h, b)
        for b in (0, 1):
            h = g + b
            wait_store(h, b)

            @pl.when(h + 2 < NCH)
            def _():
                fire_gather(h + 2, b)


def kernel(x, table):
    xf = x.astype(jnp.int32).reshape(B_TOTAL // IDXW, IDXW)
    out = _emb_lookup(xf, table)
    return out.reshape(x.shape[0], x.shape[1], D_MODEL)
